# baseline (speedup 1.0000x reference)
import functools

import jax
import jax.numpy as jnp
from jax import lax
from jax.experimental import pallas as pl
from jax.experimental.pallas import tpu as pltpu
from jax.experimental.pallas import tpu_sc as plsc

N = 10000
E = 320000
NG = 64
D = 128
BN_EPS = 1e-5

NC = 2
NS = 16
NW = NC * NS
CH = 128
GCH = 256
EPAD = 327680
EPW = EPAD // NW
NCHG = EPW // GCH
NCH = EPW // CH
NPAD = 10240
RPS = NPAD // NS
ND = 4


def _sc_aggregate(h, src3, dst3, zeros):
    mesh = plsc.VectorSubcoreMesh(
        core_axis_name="c", subcore_axis_name="s", num_cores=NC, num_subcores=NS
    )

    @functools.partial(
        pl.kernel,
        out_type=jax.ShapeDtypeStruct((NC, NPAD, D), jnp.float32),
        mesh=mesh,
        scratch_types=[
            pltpu.VMEM((2 * GCH,), jnp.int32),
            pltpu.VMEM((ND, 1, CH), jnp.int32),
            pltpu.VMEM((GCH, D), jnp.float32),
            pltpu.VMEM_SHARED((NPAD, D), jnp.float32),
        ]
        + [pltpu.SemaphoreType.DMA] * (2 + ND),
    )
    def agg_kernel(h_hbm, src_hbm, dst_hbm, z_hbm, out_hbm,
                   sidx, didx, rows, acc, *sems):
        qsem = sems[:2]
        dsem = sems[2:]
        c = lax.axis_index("c")
        s = lax.axis_index("s")
        w = c * NS + s

        pltpu.sync_copy(z_hbm.at[pl.ds(s * RPS, RPS)],
                        acc.at[pl.ds(s * RPS, RPS)])
        plsc.subcore_barrier()

        def load_sidx(j):
            off = pl.multiple_of(w * EPW + j * GCH, GCH)
            pltpu.sync_copy(src_hbm.at[pl.ds(off, GCH)],
                            sidx.at[pl.ds(0, GCH)])

        def fire_didx(q, b):
            pltpu.async_copy(dst_hbm.at[w * NCH + q], didx.at[b], dsem[b])

        def wait_didx(b):
            pltpu.make_async_copy(dst_hbm.at[0], didx.at[b], dsem[b]).wait()

        def gather():
            pltpu.sync_copy(h_hbm.at[sidx.at[pl.ds(0, GCH)]], rows)

        def scatter(b, half):
            pltpu.sync_copy(rows.at[pl.ds(half * CH, CH)],
                            acc.at[didx.at[b].at[0]], add=True)

        def chunk(j, par, prefetch):
            mine = (2 * par) % ND
            nxt = (2 * par + 2) % ND
            if prefetch:
                fire_didx(2 * j + 2, nxt)
                fire_didx(2 * j + 3, nxt + 1)
            load_sidx(j)
            gather()
            for half in (0, 1):
                wait_didx(mine + half)
                scatter(mine + half, half)

        fire_didx(0, 0)
        fire_didx(1, 1)
        chunk(0, 0, prefetch=True)

        @pl.loop(1, NCHG - 1, step=2)
        def _(g):
            chunk(g, 1, prefetch=True)
            chunk(g + 1, 0, prefetch=True)

        chunk(NCHG - 1, 1, prefetch=False)

        plsc.subcore_barrier()
        pltpu.sync_copy(acc.at[pl.ds(s * RPS, RPS)],
                        out_hbm.at[c].at[pl.ds(s * RPS, RPS)])

    return agg_kernel(h, src3, dst3, zeros)


def _tc_layer_body(h_ref, p_ref, w1_ref, b1_ref, w2_ref, b2_ref,
                   g_ref, be_ref, o_ref):
    hs = h_ref[...] + p_ref[0, :N, :] + p_ref[1, :N, :]
    a = jnp.maximum(
        jnp.dot(hs, w1_ref[...], preferred_element_type=jnp.float32)
        + b1_ref[...], 0.0)
    h2 = (jnp.dot(a, w2_ref[...], preferred_element_type=jnp.float32)
          + b2_ref[...])
    m = jnp.mean(h2, axis=0, keepdims=True)
    v = jnp.mean((h2 - m) * (h2 - m), axis=0, keepdims=True)
    o_ref[...] = jnp.maximum(
        (h2 - m) * jax.lax.rsqrt(v + BN_EPS) * g_ref[...] + be_ref[...], 0.0)


def _tc_layer(h, p, W1, b1, W2, b2, g, be):
    return pl.pallas_call(
        _tc_layer_body,
        out_shape=jax.ShapeDtypeStruct((N, D), jnp.float32),
    )(h, p, W1, b1, W2, b2, g, be)


def _tc_layer_pool_body(h_ref, p_ref, w1_ref, b1_ref, w2_ref, b2_ref,
                        g_ref, be_ref, batch_ref, o_ref):
    hs = h_ref[...] + p_ref[0, :N, :] + p_ref[1, :N, :]
    a = jnp.maximum(
        jnp.dot(hs, w1_ref[...], preferred_element_type=jnp.float32)
        + b1_ref[...], 0.0)
    h2 = (jnp.dot(a, w2_ref[...], preferred_element_type=jnp.float32)
          + b2_ref[...])
    m = jnp.mean(h2, axis=0, keepdims=True)
    v = jnp.mean((h2 - m) * (h2 - m), axis=0, keepdims=True)
    hf = jnp.maximum(
        (h2 - m) * jax.lax.rsqrt(v + BN_EPS) * g_ref[...] + be_ref[...], 0.0)
    gids = lax.broadcasted_iota(jnp.int32, (N, NG), 1)
    onehot = (batch_ref[...] == gids).astype(jnp.float32)
    sums = lax.dot_general(onehot, hf, (((0,), (0,)), ((), ())),
                           preferred_element_type=jnp.float32)
    cnt = lax.dot_general(onehot, jnp.ones((N, 1), jnp.float32),
                          (((0,), (0,)), ((), ())),
                          preferred_element_type=jnp.float32)
    o_ref[...] = sums / jnp.clip(cnt, 1.0, None)


def _tc_layer_pool(h, p, W1, b1, W2, b2, g, be, batch):
    return pl.pallas_call(
        _tc_layer_pool_body,
        out_shape=jax.ShapeDtypeStruct((NG, D), jnp.float32),
    )(h, p, W1, b1, W2, b2, g, be, batch)


def kernel(x, edge_index, batch,
           W1_0, b1_0, W2_0, b2_0, g_0, be_0,
           W1_1, b1_1, W2_1, b2_1, g_1, be_1,
           W1_2, b1_2, W2_2, b2_2, g_2, be_2):
    pad = EPAD - E
    src1 = jnp.concatenate([edge_index[0], jnp.zeros((pad,), jnp.int32)])
    dst3 = jnp.concatenate(
        [edge_index[1], jnp.full((pad,), N, jnp.int32)]).reshape(NW * NCH, 1, CH)
    zeros = jnp.zeros((NPAD, D), jnp.float32)
    batch2d = batch.reshape(N, 1)
    params = [(W1_0, b1_0, W2_0, b2_0, g_0, be_0),
              (W1_1, b1_1, W2_1, b2_1, g_1, be_1),
              (W1_2, b1_2, W2_2, b2_2, g_2, be_2)]

    h = x
    for i, (W1, b1, W2, b2, g, be) in enumerate(params):
        p = _sc_aggregate(h, src1, dst3, zeros)
        b1r = b1.reshape(1, D)
        b2r = b2.reshape(1, D)
        gr = g.reshape(1, D)
        ber = be.reshape(1, D)
        if i < 2:
            h = _tc_layer(h, p, W1, b1r, W2, b2r, gr, ber)
        else:
            h = _tc_layer_pool(h, p, W1, b1r, W2, b2r, gr, ber, batch2d)
    return h


# speedup vs baseline: 1.0004x; 1.0004x over previous
import functools

import jax
import jax.numpy as jnp
from jax import lax
from jax.experimental import pallas as pl
from jax.experimental.pallas import tpu as pltpu
from jax.experimental.pallas import tpu_sc as plsc

N = 10000
E = 320000
NG = 64
D = 128
BN_EPS = 1e-5

NC = 2
NS = 16
NW = NC * NS
CH = 128
GCH = 256
EPAD = 327680
EPW = EPAD // NW
NCHG = EPW // GCH
NCH = EPW // CH
NPAD = 10240
RPS = NPAD // NS
ND = 4


def _sc_aggregate(h, src3, dst3, zeros):
    mesh = plsc.VectorSubcoreMesh(
        core_axis_name="c", subcore_axis_name="s", num_cores=NC, num_subcores=NS
    )

    @functools.partial(
        pl.kernel,
        out_type=jax.ShapeDtypeStruct((NC, NPAD, D), jnp.float32),
        mesh=mesh,
        scratch_types=[
            pltpu.VMEM((2 * GCH,), jnp.int32),
            pltpu.VMEM((ND, 1, CH), jnp.int32),
            pltpu.VMEM((GCH, D), jnp.float32),
            pltpu.VMEM_SHARED((NPAD, D), jnp.float32),
        ]
        + [pltpu.SemaphoreType.DMA] * (2 + ND),
    )
    def agg_kernel(h_hbm, src_hbm, dst_hbm, z_hbm, out_hbm,
                   sidx, didx, rows, acc, *sems):
        qsem = sems[:2]
        dsem = sems[2:]
        c = lax.axis_index("c")
        s = lax.axis_index("s")
        w = c * NS + s

        pltpu.sync_copy(z_hbm.at[pl.ds(s * RPS, RPS)],
                        acc.at[pl.ds(s * RPS, RPS)])
        plsc.subcore_barrier()

        def load_sidx(j):
            off = pl.multiple_of(w * EPW + j * GCH, GCH)
            pltpu.sync_copy(src_hbm.at[pl.ds(off, GCH)],
                            sidx.at[pl.ds(0, GCH)])

        def fire_didx(q, b):
            pltpu.async_copy(dst_hbm.at[w * NCH + q], didx.at[b], dsem[b])

        def wait_didx(b):
            pltpu.make_async_copy(dst_hbm.at[0], didx.at[b], dsem[b]).wait()

        def gather():
            pltpu.sync_copy(h_hbm.at[sidx.at[pl.ds(0, GCH)]], rows)

        def scatter(b, half):
            pltpu.sync_copy(rows.at[pl.ds(half * CH, CH)],
                            acc.at[didx.at[b].at[0]], add=True)

        def chunk(j, par, prefetch):
            mine = (2 * par) % ND
            nxt = (2 * par + 2) % ND
            if prefetch:
                fire_didx(2 * j + 2, nxt)
                fire_didx(2 * j + 3, nxt + 1)
            load_sidx(j)
            gather()
            for half in (0, 1):
                wait_didx(mine + half)
                scatter(mine + half, half)

        fire_didx(0, 0)
        fire_didx(1, 1)
        chunk(0, 0, prefetch=True)

        @pl.loop(1, NCHG - 1, step=2)
        def _(g):
            chunk(g, 1, prefetch=True)
            chunk(g + 1, 0, prefetch=True)

        chunk(NCHG - 1, 1, prefetch=False)

        plsc.subcore_barrier()
        pltpu.sync_copy(acc.at[pl.ds(s * RPS, RPS)],
                        out_hbm.at[c].at[pl.ds(s * RPS, RPS)])

    return agg_kernel(h, src3, dst3, zeros)


def _tc_layer_body(h_ref, p_ref, w1_ref, b1_ref, w2_ref, b2_ref,
                   g_ref, be_ref, o_ref):
    hs = h_ref[...] + p_ref[0, :N, :] + p_ref[1, :N, :]
    a = jnp.maximum(
        jnp.dot(hs, w1_ref[...], preferred_element_type=jnp.float32)
        + b1_ref[...], 0.0)
    h2 = (jnp.dot(a, w2_ref[...], preferred_element_type=jnp.float32)
          + b2_ref[...])
    m = jnp.mean(h2, axis=0, keepdims=True)
    v = jnp.mean((h2 - m) * (h2 - m), axis=0, keepdims=True)
    o_ref[...] = jnp.maximum(
        (h2 - m) * jax.lax.rsqrt(v + BN_EPS) * g_ref[...] + be_ref[...], 0.0)


def _tc_layer(h, p, W1, b1, W2, b2, g, be):
    return pl.pallas_call(
        _tc_layer_body,
        out_shape=jax.ShapeDtypeStruct((N, D), jnp.float32),
    )(h, p, W1, b1, W2, b2, g, be)


def _tc_layer_pool_body(h_ref, p_ref, w1_ref, b1_ref, w2_ref, b2_ref,
                        g_ref, be_ref, batch_ref, o_ref):
    hs = h_ref[...] + p_ref[0, :N, :] + p_ref[1, :N, :]
    a = jnp.maximum(
        jnp.dot(hs, w1_ref[...], preferred_element_type=jnp.float32)
        + b1_ref[...], 0.0)
    h2 = (jnp.dot(a, w2_ref[...], preferred_element_type=jnp.float32)
          + b2_ref[...])
    m = jnp.mean(h2, axis=0, keepdims=True)
    v = jnp.mean((h2 - m) * (h2 - m), axis=0, keepdims=True)
    hf = jnp.maximum(
        (h2 - m) * jax.lax.rsqrt(v + BN_EPS) * g_ref[...] + be_ref[...], 0.0)
    gids = lax.broadcasted_iota(jnp.int32, (N, NG), 1)
    onehot = (batch_ref[...] == gids).astype(jnp.float32)
    sums = lax.dot_general(onehot, hf, (((0,), (0,)), ((), ())),
                           preferred_element_type=jnp.float32)
    cnt = lax.dot_general(onehot, jnp.ones((N, 1), jnp.float32),
                          (((0,), (0,)), ((), ())),
                          preferred_element_type=jnp.float32)
    o_ref[...] = sums / jnp.clip(cnt, 1.0, None)


def _tc_layer_pool(h, p, W1, b1, W2, b2, g, be, batch):
    return pl.pallas_call(
        _tc_layer_pool_body,
        out_shape=jax.ShapeDtypeStruct((NG, D), jnp.float32),
    )(h, p, W1, b1, W2, b2, g, be, batch)


def kernel(x, edge_index, batch,
           W1_0, b1_0, W2_0, b2_0, g_0, be_0,
           W1_1, b1_1, W2_1, b2_1, g_1, be_1,
           W1_2, b1_2, W2_2, b2_2, g_2, be_2):
    pad = EPAD - E
    src1 = jnp.concatenate([edge_index[0], jnp.zeros((pad,), jnp.int32)])
    junk = N + jnp.arange(pad, dtype=jnp.int32) % (NPAD - N)
    dst3 = jnp.concatenate([edge_index[1], junk]).reshape(NW * NCH, 1, CH)
    zeros = jnp.zeros((NPAD, D), jnp.float32)
    batch2d = batch.reshape(N, 1)
    params = [(W1_0, b1_0, W2_0, b2_0, g_0, be_0),
              (W1_1, b1_1, W2_1, b2_1, g_1, be_1),
              (W1_2, b1_2, W2_2, b2_2, g_2, be_2)]

    h = x
    for i, (W1, b1, W2, b2, g, be) in enumerate(params):
        p = _sc_aggregate(h, src1, dst3, zeros)
        b1r = b1.reshape(1, D)
        b2r = b2.reshape(1, D)
        gr = g.reshape(1, D)
        ber = be.reshape(1, D)
        if i < 2:
            h = _tc_layer(h, p, W1, b1r, W2, b2r, gr, ber)
        else:
            h = _tc_layer_pool(h, p, W1, b1r, W2, b2r, gr, ber, batch2d)
    return h


# speedup vs baseline: 2.9901x; 2.9888x over previous
import functools

import jax
import jax.numpy as jnp
from jax import lax
from jax.experimental import pallas as pl
from jax.experimental.pallas import tpu as pltpu
from jax.experimental.pallas import tpu_sc as plsc

N = 10000
E = 320000
NG = 64
D = 128
BN_EPS = 1e-5

NC = 2
NS = 16
NW = NC * NS
CH = 128
EPAD = 327680
EPW = EPAD // NW
NCH = EPW // CH
NPAD = 10240
RPS = NPAD // NS


def _sc_aggregate(h, src1, dst1, zeros):
    mesh = plsc.VectorSubcoreMesh(
        core_axis_name="c", subcore_axis_name="s", num_cores=NC, num_subcores=NS
    )

    @functools.partial(
        pl.kernel,
        out_type=jax.ShapeDtypeStruct((NC, NPAD, D), jnp.float32),
        mesh=mesh,
        scratch_types=[
            pltpu.VMEM((16, CH), jnp.int32),
            pltpu.VMEM((16, CH), jnp.int32),
            pltpu.VMEM((CH, D), jnp.float32),
            pltpu.VMEM_SHARED((NPAD, D), jnp.float32),
        ]
        + [pltpu.SemaphoreType.DMA] * 4,
    )
    def agg_kernel(h_hbm, src_hbm, dst_hbm, z_hbm, out_hbm,
                   sidx, didx, rows, acc, *sems):
        qs = sems[:2]
        qd = sems[2:]
        c = lax.axis_index("c")
        s = lax.axis_index("s")
        w = c * NS + s

        pltpu.sync_copy(z_hbm.at[pl.ds(s * RPS, RPS)],
                        acc.at[pl.ds(s * RPS, RPS)])
        plsc.subcore_barrier()

        def fire_idx(j, b):
            off = pl.multiple_of(w * EPW, CH) + j * CH
            pltpu.async_copy(src_hbm.at[pl.ds(off, CH)], sidx.at[8 * b], qs[b])
            pltpu.async_copy(dst_hbm.at[pl.ds(off, CH)], didx.at[8 * b], qd[b])

        def wait_idx(b):
            pltpu.make_async_copy(src_hbm.at[pl.ds(0, CH)], sidx.at[8 * b],
                                  qs[b]).wait()
            pltpu.make_async_copy(dst_hbm.at[pl.ds(0, CH)], didx.at[8 * b],
                                  qd[b]).wait()

        def chunk(j, par, prefetch):
            if prefetch:
                fire_idx(j + 1, 1 - par)
            wait_idx(par)
            pltpu.sync_copy(h_hbm.at[sidx.at[8 * par]], rows)
            pltpu.sync_copy(rows, acc.at[didx.at[8 * par]], add=True)

        fire_idx(0, 0)
        chunk(0, 0, prefetch=True)

        @pl.loop(1, NCH - 1, step=2)
        def _(g):
            chunk(g, 1, prefetch=True)
            chunk(g + 1, 0, prefetch=True)

        chunk(NCH - 1, 1, prefetch=False)

        plsc.subcore_barrier()
        pltpu.sync_copy(acc.at[pl.ds(s * RPS, RPS)],
                        out_hbm.at[c].at[pl.ds(s * RPS, RPS)])

    return agg_kernel(h, src1, dst1, zeros)


def _tc_layer_body(h_ref, p_ref, w1_ref, b1_ref, w2_ref, b2_ref,
                   g_ref, be_ref, o_ref):
    hs = h_ref[...] + p_ref[0, :N, :] + p_ref[1, :N, :]
    a = jnp.maximum(
        jnp.dot(hs, w1_ref[...], preferred_element_type=jnp.float32)
        + b1_ref[...], 0.0)
    h2 = (jnp.dot(a, w2_ref[...], preferred_element_type=jnp.float32)
          + b2_ref[...])
    m = jnp.mean(h2, axis=0, keepdims=True)
    v = jnp.mean((h2 - m) * (h2 - m), axis=0, keepdims=True)
    o_ref[...] = jnp.maximum(
        (h2 - m) * jax.lax.rsqrt(v + BN_EPS) * g_ref[...] + be_ref[...], 0.0)


def _tc_layer(h, p, W1, b1, W2, b2, g, be):
    return pl.pallas_call(
        _tc_layer_body,
        out_shape=jax.ShapeDtypeStruct((N, D), jnp.float32),
    )(h, p, W1, b1, W2, b2, g, be)


def _tc_layer_pool_body(h_ref, p_ref, w1_ref, b1_ref, w2_ref, b2_ref,
                        g_ref, be_ref, batch_ref, o_ref):
    hs = h_ref[...] + p_ref[0, :N, :] + p_ref[1, :N, :]
    a = jnp.maximum(
        jnp.dot(hs, w1_ref[...], preferred_element_type=jnp.float32)
        + b1_ref[...], 0.0)
    h2 = (jnp.dot(a, w2_ref[...], preferred_element_type=jnp.float32)
          + b2_ref[...])
    m = jnp.mean(h2, axis=0, keepdims=True)
    v = jnp.mean((h2 - m) * (h2 - m), axis=0, keepdims=True)
    hf = jnp.maximum(
        (h2 - m) * jax.lax.rsqrt(v + BN_EPS) * g_ref[...] + be_ref[...], 0.0)
    gids = lax.broadcasted_iota(jnp.int32, (N, NG), 1)
    onehot = (batch_ref[...] == gids).astype(jnp.float32)
    sums = lax.dot_general(onehot, hf, (((0,), (0,)), ((), ())),
                           preferred_element_type=jnp.float32)
    cnt = lax.dot_general(onehot, jnp.ones((N, 1), jnp.float32),
                          (((0,), (0,)), ((), ())),
                          preferred_element_type=jnp.float32)
    o_ref[...] = sums / jnp.clip(cnt, 1.0, None)


def _tc_layer_pool(h, p, W1, b1, W2, b2, g, be, batch):
    return pl.pallas_call(
        _tc_layer_pool_body,
        out_shape=jax.ShapeDtypeStruct((NG, D), jnp.float32),
    )(h, p, W1, b1, W2, b2, g, be, batch)


def kernel(x, edge_index, batch,
           W1_0, b1_0, W2_0, b2_0, g_0, be_0,
           W1_1, b1_1, W2_1, b2_1, g_1, be_1,
           W1_2, b1_2, W2_2, b2_2, g_2, be_2):
    pad = EPAD - E
    psrc = jnp.arange(pad, dtype=jnp.int32) % N
    pdst = N + jnp.arange(pad, dtype=jnp.int32) % (NPAD - N)
    src1 = jnp.concatenate([edge_index[0], psrc])
    dst1 = jnp.concatenate([edge_index[1], pdst])
    zeros = jnp.zeros((NPAD, D), jnp.float32)
    batch2d = batch.reshape(N, 1)
    params = [(W1_0, b1_0, W2_0, b2_0, g_0, be_0),
              (W1_1, b1_1, W2_1, b2_1, g_1, be_1),
              (W1_2, b1_2, W2_2, b2_2, g_2, be_2)]

    h = x
    for i, (W1, b1, W2, b2, g, be) in enumerate(params):
        p = _sc_aggregate(h, src1, dst1, zeros)
        b1r = b1.reshape(1, D)
        b2r = b2.reshape(1, D)
        gr = g.reshape(1, D)
        ber = be.reshape(1, D)
        if i < 2:
            h = _tc_layer(h, p, W1, b1r, W2, b2r, gr, ber)
        else:
            h = _tc_layer_pool(h, p, W1, b1r, W2, b2r, gr, ber, batch2d)
    return h


# speedup vs baseline: 3.8833x; 1.2987x over previous
import functools

import jax
import jax.numpy as jnp
from jax import lax
from jax.experimental import pallas as pl
from jax.experimental.pallas import tpu as pltpu
from jax.experimental.pallas import tpu_sc as plsc

N = 10000
E = 320000
NG = 64
D = 128
BN_EPS = 1e-5

NC = 2
NS = 16
NW = NC * NS
CH = 128
EPAD = 327680
EPW = EPAD // NW
NCH = EPW // CH
NPAD = 10240
RPS = NPAD // NS


def _sc_aggregate(h, src1, dst1, zeros):
    mesh = plsc.VectorSubcoreMesh(
        core_axis_name="c", subcore_axis_name="s", num_cores=NC, num_subcores=NS
    )

    @functools.partial(
        pl.kernel,
        out_type=jax.ShapeDtypeStruct((NC, NPAD, D), jnp.float32),
        mesh=mesh,
        scratch_types=[
            pltpu.VMEM((32, CH), jnp.int32),
            pltpu.VMEM((32, CH), jnp.int32),
            pltpu.VMEM((CH, D), jnp.float32),
            pltpu.VMEM((CH, D), jnp.float32),
            pltpu.VMEM_SHARED((NPAD, D), jnp.float32),
        ]
        + [pltpu.SemaphoreType.DMA] * 12,
    )
    def agg_kernel(h_hbm, src_hbm, dst_hbm, z_hbm, out_hbm,
                   sidx, didx, rows0, rows1, acc, *sems):
        qs = sems[:4]
        qd = sems[4:8]
        gsem = sems[8:10]
        ssem = sems[10:]
        rows = (rows0, rows1)
        c = lax.axis_index("c")
        s = lax.axis_index("s")
        w = c * NS + s

        pltpu.sync_copy(z_hbm.at[pl.ds(s * RPS, RPS)],
                        acc.at[pl.ds(s * RPS, RPS)])
        plsc.subcore_barrier()

        def fire_idx(j, b):
            off = pl.multiple_of(w * EPW, CH) + j * CH
            pltpu.async_copy(src_hbm.at[pl.ds(off, CH)], sidx.at[8 * b], qs[b])
            pltpu.async_copy(dst_hbm.at[pl.ds(off, CH)], didx.at[8 * b], qd[b])

        def wait_idx(b):
            pltpu.make_async_copy(src_hbm.at[pl.ds(0, CH)], sidx.at[8 * b],
                                  qs[b]).wait()
            pltpu.make_async_copy(dst_hbm.at[pl.ds(0, CH)], didx.at[8 * b],
                                  qd[b]).wait()

        def fire_gather(j, q, r):
            pltpu.async_copy(h_hbm.at[sidx.at[8 * q]], rows[r], gsem[r])

        def wait_gather(r):
            pltpu.make_async_copy(h_hbm.at[sidx.at[0]], rows[r],
                                  gsem[r]).wait()

        def fire_scatter(q, r):
            pltpu.async_copy(rows[r], acc.at[didx.at[8 * q]], ssem[r],
                             add=True)

        def wait_scatter(r):
            pltpu.make_async_copy(rows[r], acc.at[didx.at[0]], ssem[r]).wait()

        def body(j, r, q, first=False, last=False, more=True):
            wait_gather(r)
            fire_scatter(q, r)
            if not first:
                wait_scatter(1 - r)
            if more:
                fire_idx(j + 2, (q + 2) % 4)
            if not last:
                wait_idx((q + 1) % 4)
                fire_gather(j + 1, (q + 1) % 4, 1 - r)

        fire_idx(0, 0)
        fire_idx(1, 1)
        wait_idx(0)
        fire_gather(0, 0, 0)

        body(0, 0, 0, first=True)
        body(1, 1, 1)

        @pl.loop(2, NCH - 2, step=4)
        def _(g):
            for u in range(4):
                body(g + u, u % 2, (2 + u) % 4)

        body(NCH - 2, 0, 2, more=False)
        body(NCH - 1, 1, 3, last=True, more=False)
        wait_scatter(1)

        plsc.subcore_barrier()
        pltpu.sync_copy(acc.at[pl.ds(s * RPS, RPS)],
                        out_hbm.at[c].at[pl.ds(s * RPS, RPS)])

    return agg_kernel(h, src1, dst1, zeros)


def _tc_layer_body(h_ref, p_ref, w1_ref, b1_ref, w2_ref, b2_ref,
                   g_ref, be_ref, o_ref):
    hs = h_ref[...] + p_ref[0, :N, :] + p_ref[1, :N, :]
    a = jnp.maximum(
        jnp.dot(hs, w1_ref[...], preferred_element_type=jnp.float32)
        + b1_ref[...], 0.0)
    h2 = (jnp.dot(a, w2_ref[...], preferred_element_type=jnp.float32)
          + b2_ref[...])
    m = jnp.mean(h2, axis=0, keepdims=True)
    v = jnp.mean((h2 - m) * (h2 - m), axis=0, keepdims=True)
    o_ref[...] = jnp.maximum(
        (h2 - m) * jax.lax.rsqrt(v + BN_EPS) * g_ref[...] + be_ref[...], 0.0)


def _tc_layer(h, p, W1, b1, W2, b2, g, be):
    return pl.pallas_call(
        _tc_layer_body,
        out_shape=jax.ShapeDtypeStruct((N, D), jnp.float32),
    )(h, p, W1, b1, W2, b2, g, be)


def _tc_layer_pool_body(h_ref, p_ref, w1_ref, b1_ref, w2_ref, b2_ref,
                        g_ref, be_ref, batch_ref, o_ref):
    hs = h_ref[...] + p_ref[0, :N, :] + p_ref[1, :N, :]
    a = jnp.maximum(
        jnp.dot(hs, w1_ref[...], preferred_element_type=jnp.float32)
        + b1_ref[...], 0.0)
    h2 = (jnp.dot(a, w2_ref[...], preferred_element_type=jnp.float32)
          + b2_ref[...])
    m = jnp.mean(h2, axis=0, keepdims=True)
    v = jnp.mean((h2 - m) * (h2 - m), axis=0, keepdims=True)
    hf = jnp.maximum(
        (h2 - m) * jax.lax.rsqrt(v + BN_EPS) * g_ref[...] + be_ref[...], 0.0)
    gids = lax.broadcasted_iota(jnp.int32, (N, NG), 1)
    onehot = (batch_ref[...] == gids).astype(jnp.float32)
    sums = lax.dot_general(onehot, hf, (((0,), (0,)), ((), ())),
                           preferred_element_type=jnp.float32)
    cnt = lax.dot_general(onehot, jnp.ones((N, 1), jnp.float32),
                          (((0,), (0,)), ((), ())),
                          preferred_element_type=jnp.float32)
    o_ref[...] = sums / jnp.clip(cnt, 1.0, None)


def _tc_layer_pool(h, p, W1, b1, W2, b2, g, be, batch):
    return pl.pallas_call(
        _tc_layer_pool_body,
        out_shape=jax.ShapeDtypeStruct((NG, D), jnp.float32),
    )(h, p, W1, b1, W2, b2, g, be, batch)


def kernel(x, edge_index, batch,
           W1_0, b1_0, W2_0, b2_0, g_0, be_0,
           W1_1, b1_1, W2_1, b2_1, g_1, be_1,
           W1_2, b1_2, W2_2, b2_2, g_2, be_2):
    pad = EPAD - E
    psrc = jnp.arange(pad, dtype=jnp.int32) % N
    pdst = N + jnp.arange(pad, dtype=jnp.int32) % (NPAD - N)
    src1 = jnp.concatenate([edge_index[0], psrc])
    dst1 = jnp.concatenate([edge_index[1], pdst])
    zeros = jnp.zeros((NPAD, D), jnp.float32)
    batch2d = batch.reshape(N, 1)
    params = [(W1_0, b1_0, W2_0, b2_0, g_0, be_0),
              (W1_1, b1_1, W2_1, b2_1, g_1, be_1),
              (W1_2, b1_2, W2_2, b2_2, g_2, be_2)]

    h = x
    for i, (W1, b1, W2, b2, g, be) in enumerate(params):
        p = _sc_aggregate(h, src1, dst1, zeros)
        b1r = b1.reshape(1, D)
        b2r = b2.reshape(1, D)
        gr = g.reshape(1, D)
        ber = be.reshape(1, D)
        if i < 2:
            h = _tc_layer(h, p, W1, b1r, W2, b2r, gr, ber)
        else:
            h = _tc_layer_pool(h, p, W1, b1r, W2, b2r, gr, ber, batch2d)
    return h
